# baseline (speedup 1.0000x reference)
import jax
import jax.numpy as jnp
from jax import lax
from jax.experimental import pallas as pl
from jax.experimental.pallas import tpu as pltpu
from jax.experimental.pallas import tpu_sc as plsc

N = 10000
NPAD = 10240
E = 320000
D = 128
F = 64
NC = 2
NS = 16
EPAD = 327680
ER = E // 128
EPR = EPAD // 128
RT = EPR // NS
HB = RT // 2
NBUF = 5
PRE = 3
RP = NPAD // NS
THIRD = 1.0 / 3.0
_PREC = jax.lax.Precision.HIGHEST


def _pre_body(x_ref, eit_ref, eir_in_ref, wt_ref, wrel_ref,
              wst_ref, wnt_ref, wsr_ref, wnr_ref, w1_ref,
              z_ref, y2_ref, eir_ref, wr_ref, ipad_ref, fpad_ref, sem):
    ipad_ref[...] = jnp.full((2, EPR - ER, 128), N, jnp.int32)
    fpad_ref[...] = jnp.zeros((EPR - ER, 128), jnp.float32)
    cps = [
        pltpu.make_async_copy(eit_ref, eir_ref.at[0, :, pl.ds(0, ER)], sem),
        pltpu.make_async_copy(eir_in_ref, eir_ref.at[1, :, pl.ds(0, ER)], sem),
        pltpu.make_async_copy(ipad_ref, eir_ref.at[0, :, pl.ds(ER, EPR - ER)],
                              sem),
        pltpu.make_async_copy(ipad_ref, eir_ref.at[1, :, pl.ds(ER, EPR - ER)],
                              sem),
        pltpu.make_async_copy(wt_ref, wr_ref.at[0, pl.ds(0, ER)], sem),
        pltpu.make_async_copy(wrel_ref, wr_ref.at[1, pl.ds(0, ER)], sem),
        pltpu.make_async_copy(fpad_ref, wr_ref.at[0, pl.ds(ER, EPR - ER)],
                              sem),
        pltpu.make_async_copy(fpad_ref, wr_ref.at[1, pl.ds(ER, EPR - ER)],
                              sem),
    ]
    for cp in cps:
        cp.start()
    w1 = w1_ref[...]
    x = x_ref[...]
    at = jnp.dot(wnt_ref[...], w1, precision=_PREC) * THIRD
    ar = jnp.dot(wnr_ref[...], w1, precision=_PREC) * THIRD
    eye = (lax.broadcasted_iota(jnp.int32, (D, D), 0)
           == lax.broadcasted_iota(jnp.int32, (D, D), 1)).astype(jnp.float32)
    az = jnp.dot(wst_ref[...] + wsr_ref[...] + eye, w1, precision=_PREC) * THIRD
    zpad = jnp.zeros((NPAD - N, F), jnp.float32)
    y2_ref[pl.ds(0, N)] = jnp.dot(x, at, precision=_PREC)
    y2_ref[pl.ds(N, NPAD - N)] = zpad
    y2_ref[pl.ds(NPAD, N)] = jnp.dot(x, ar, precision=_PREC)
    y2_ref[pl.ds(NPAD + N, NPAD - N)] = zpad
    z_ref[pl.ds(0, N)] = jnp.dot(x, az, precision=_PREC)
    z_ref[pl.ds(N, NPAD - N)] = zpad
    for cp in cps:
        cp.wait()


_pre_call = pl.pallas_call(
    _pre_body,
    in_specs=[pl.BlockSpec(memory_space=pltpu.VMEM),
              pl.BlockSpec(memory_space=pl.ANY),
              pl.BlockSpec(memory_space=pl.ANY),
              pl.BlockSpec(memory_space=pl.ANY),
              pl.BlockSpec(memory_space=pl.ANY)]
             + [pl.BlockSpec(memory_space=pltpu.VMEM)] * 5,
    out_specs=[pl.BlockSpec(memory_space=pltpu.VMEM),
               pl.BlockSpec(memory_space=pltpu.VMEM),
               pl.BlockSpec(memory_space=pl.ANY),
               pl.BlockSpec(memory_space=pl.ANY)],
    out_shape=[jax.ShapeDtypeStruct((NPAD, F), jnp.float32),
               jax.ShapeDtypeStruct((2 * NPAD, F), jnp.float32),
               jax.ShapeDtypeStruct((NC, 2, EPR, 128), jnp.int32),
               jax.ShapeDtypeStruct((NC, EPR, 128), jnp.float32)],
    scratch_shapes=[pltpu.VMEM((2, EPR - ER, 128), jnp.int32),
                    pltpu.VMEM((EPR - ER, 128), jnp.float32),
                    pltpu.SemaphoreType.DMA],
)


def _sc_body(y2, eir, wr, s_out, deg_out,
             src_blk, dst_blk, w_blk, rows_v, ones_v, deg_acc,
             acc_sh, degacc_sh, semg, sems):
    cid = lax.axis_index("c")
    sid = lax.axis_index("s")
    row0 = sid * RP
    irow0 = sid * RT
    zero16 = jnp.zeros((16,), jnp.float32)
    one16 = jnp.ones((16,), jnp.float32)
    yoff = cid * NPAD

    def _fire(kk, b):
        pltpu.async_copy(y2.at[src_blk.at[kk]], rows_v.at[b], semg.at[b])

    def _wait_gathers(b):
        pltpu.make_async_copy(y2.at[pl.ds(0, 128)], rows_v.at[b],
                              semg.at[b]).wait()

    def _scale(kk, b):
        def _grp(g, _, b=b):
            wv = w_blk[kk, pl.ds(g * 16, 16)]
            for jj in range(16):
                w = wv[jj]
                e = g * 16 + jj
                for q in range(F // 16):
                    sl = pl.ds(q * 16, 16)
                    rows_v[b, e, sl] = rows_v[b, e, sl] * w
            return 0
        lax.fori_loop(0, 8, _grp, 0)

    def _fire_scatter(kk, b):
        pltpu.async_copy(rows_v.at[b], acc_sh.at[dst_blk.at[kk]],
                         sems.at[b], add=True)
        pltpu.async_copy(ones_v, degacc_sh.at[dst_blk.at[kk]],
                         sems.at[b], add=True)

    def _wait_scatter(b):
        pltpu.make_async_copy(y2.at[pl.ds(0, 128)], rows_v.at[b],
                              sems.at[b]).wait()
        pltpu.make_async_copy(wr.at[cid, 0], ones_v, sems.at[b]).wait()

    for h in range(2):
        hrow0 = irow0 + h * HB
        pltpu.async_copy(eir.at[cid, 0, pl.ds(hrow0, HB)], src_blk,
                         semg.at[0])
        pltpu.async_copy(eir.at[cid, 1, pl.ds(hrow0, HB)], dst_blk,
                         semg.at[1])
        pltpu.async_copy(wr.at[cid, pl.ds(hrow0, HB)], w_blk, semg.at[2])

        if h == 0:
            for i in range(8):
                ones_v[pl.ds(i * 16, 16)] = one16

            def _z_rows(i, _):
                for b in range(NBUF):
                    for q in range(F // 16):
                        rows_v[b, i, pl.ds(q * 16, 16)] = zero16
                return 0
            lax.fori_loop(0, 128, _z_rows, 0)

            def _z_dacc(i, _):
                deg_acc[pl.ds(i * 16, 16)] = zero16
                return 0
            lax.fori_loop(0, RP // 16, _z_dacc, 0)

            for b in range(NBUF):
                pltpu.sync_copy(rows_v.at[b],
                                acc_sh.at[pl.ds(row0 + b * 128, 128)])
            pltpu.sync_copy(deg_acc, degacc_sh.at[pl.ds(row0, RP)])

        pltpu.make_async_copy(eir.at[cid, 0, pl.ds(0, HB)], src_blk,
                              semg.at[0]).wait()
        pltpu.make_async_copy(eir.at[cid, 0, pl.ds(0, HB)], dst_blk,
                              semg.at[1]).wait()
        pltpu.make_async_copy(wr.at[cid, pl.ds(0, HB)], w_blk,
                              semg.at[2]).wait()

        def _off(i, _):
            for t in range(8):
                sl = pl.ds(t * 16, 16)
                src_blk[i, sl] = src_blk[i, sl] + yoff
            return 0
        lax.fori_loop(0, HB, _off, 0)

        if h == 0:
            plsc.subcore_barrier()

        for b in range(PRE):
            _fire(b, b)

        def _step(k, _):
            for b in range(NBUF):
                kk = k * NBUF + b
                _wait_gathers(b)
                _scale(kk, b)
                _fire_scatter(kk, b)
                kk2 = kk + PRE
                b2 = (b + PRE) % NBUF

                @pl.when(kk2 < HB)
                def _():
                    @pl.when(kk2 >= NBUF)
                    def _():
                        _wait_scatter(b2)
                    _fire(kk2, b2)
            return 0

        lax.fori_loop(0, HB // NBUF, _step, 0)
        for b in range(NBUF):
            _wait_scatter(b)

    plsc.subcore_barrier()
    for b in range(NBUF):
        pltpu.sync_copy(acc_sh.at[pl.ds(row0 + b * 128, 128)], rows_v.at[b])
        pltpu.sync_copy(rows_v.at[b], s_out.at[cid, pl.ds(row0 + b * 128, 128)])
    pltpu.sync_copy(degacc_sh.at[pl.ds(row0, RP)], deg_acc)
    pltpu.sync_copy(deg_acc, deg_out.at[cid, pl.ds(row0, RP)])


_sc_call = pl.kernel(
    _sc_body,
    out_type=(jax.ShapeDtypeStruct((NC, NPAD, F), jnp.float32),
              jax.ShapeDtypeStruct((NC, NPAD), jnp.float32)),
    mesh=plsc.VectorSubcoreMesh(core_axis_name="c", subcore_axis_name="s"),
    compiler_params=pltpu.CompilerParams(needs_layout_passes=False,
                                         use_tc_tiling_on_sc=False),
    scratch_types=[
        pltpu.VMEM((HB, 128), jnp.int32),
        pltpu.VMEM((HB, 128), jnp.int32),
        pltpu.VMEM((HB, 128), jnp.float32),
        pltpu.VMEM((NBUF, 128, F), jnp.float32),
        pltpu.VMEM((128,), jnp.float32),
        pltpu.VMEM((RP,), jnp.float32),
        pltpu.VMEM_SHARED((NPAD, F), jnp.float32),
        pltpu.VMEM_SHARED((NPAD,), jnp.float32),
        pltpu.SemaphoreType.DMA((NBUF,)),
        pltpu.SemaphoreType.DMA((NBUF,)),
    ],
)


def _post_body(z_ref, s_ref, deg_ref, w1_ref, b1_ref, bt_ref, br_ref,
               w2_ref, b2_ref, w3_ref, b3_ref, out_ref):
    c = b1_ref[...] + jnp.dot(bt_ref[...] + br_ref[...], w1_ref[...],
                              precision=_PREC) * THIRD
    d0 = lax.broadcast_in_dim(jnp.maximum(deg_ref[0], 1.0), (NPAD, F), (0,))
    d1 = lax.broadcast_in_dim(jnp.maximum(deg_ref[1], 1.0), (NPAD, F), (0,))
    h1 = z_ref[...] + s_ref[0] / d0 + s_ref[1] / d1 + c
    h1 = jnp.maximum(h1, 0.0)
    h2 = jnp.maximum(jnp.dot(h1, w2_ref[...], precision=_PREC) + b2_ref[...],
                     0.0)
    out = jnp.dot(h2, w3_ref[...], precision=_PREC) + b3_ref[...]
    out_ref[...] = out[:N]


_post_call = pl.pallas_call(
    _post_body,
    out_shape=jax.ShapeDtypeStruct((N, 10), jnp.float32),
)


def kernel(x, edge_index_sim_tic, edge_weight_sim_tic,
           edge_index_related_to, edge_weight_related_to,
           W_self_tic, W_neigh_tic, b_tic,
           W_self_rel, W_neigh_rel, b_rel,
           W1, b1, W2, b2, W3, b3):
    z, y2, eir, wr = _pre_call(
        x,
        edge_index_sim_tic.reshape(2, ER, 128),
        edge_index_related_to.reshape(2, ER, 128),
        edge_weight_sim_tic.reshape(ER, 128),
        edge_weight_related_to.reshape(ER, 128),
        W_self_tic, W_neigh_tic, W_self_rel, W_neigh_rel, W1)
    s, deg = _sc_call(y2, eir, wr)
    out = _post_call(z, s, deg,
                     W1, b1.reshape(1, F), b_tic.reshape(1, D),
                     b_rel.reshape(1, D), W2, b2.reshape(1, 32),
                     W3, b3.reshape(1, 10))
    return (out, out)


# speedup vs baseline: 1.4938x; 1.4938x over previous
import jax
import jax.numpy as jnp
from jax import lax
from jax.experimental import pallas as pl
from jax.experimental.pallas import tpu as pltpu
from jax.experimental.pallas import tpu_sc as plsc

N = 10000
NPAD = 10240
E = 320000
D = 128
F = 64
NC = 2
NS = 16
EPAD = 327680
ER = E // 128
EPR = EPAD // 128
RT = EPR // NS
QB = RT // 4
KR = 2
NKH = QB // KR
NBUF = 4
PRE = 2
RP = NPAD // NS
THIRD = 1.0 / 3.0
_PREC = jax.lax.Precision.HIGHEST


def _pre_body(x_ref, wst_ref, wnt_ref, wsr_ref, wnr_ref, w1_ref,
              z_ref, y2_ref):
    w1 = w1_ref[...]
    x = x_ref[...]
    at = jnp.dot(wnt_ref[...], w1, precision=_PREC) * THIRD
    ar = jnp.dot(wnr_ref[...], w1, precision=_PREC) * THIRD
    eye = (lax.broadcasted_iota(jnp.int32, (D, D), 0)
           == lax.broadcasted_iota(jnp.int32, (D, D), 1)).astype(jnp.float32)
    az = jnp.dot(wst_ref[...] + wsr_ref[...] + eye, w1, precision=_PREC) * THIRD
    zpad = jnp.zeros((NPAD - N, F), jnp.float32)
    y2_ref[pl.ds(0, N)] = jnp.dot(x, at, precision=_PREC)
    y2_ref[pl.ds(N, NPAD - N)] = zpad
    y2_ref[pl.ds(NPAD, N)] = jnp.dot(x, ar, precision=_PREC)
    y2_ref[pl.ds(NPAD + N, NPAD - N)] = zpad
    z_ref[pl.ds(0, N)] = jnp.dot(x, az, precision=_PREC)
    z_ref[pl.ds(N, NPAD - N)] = zpad


_pre_call = pl.pallas_call(
    _pre_body,
    out_shape=[jax.ShapeDtypeStruct((NPAD, F), jnp.float32),
               jax.ShapeDtypeStruct((2 * NPAD, F), jnp.float32)],
)


def _sc_body(y2, srcs, dsts, wr, s_out, deg_out,
             src_blk, dst_blk, w_blk, rows_v, ones_v, deg_acc,
             acc_sh, degacc_sh, semg, sems):
    cid = lax.axis_index("c")
    sid = lax.axis_index("s")
    row0 = sid * RP
    irow0 = sid * RT
    zero16 = jnp.zeros((16,), jnp.float32)
    one16 = jnp.ones((16,), jnp.float32)
    yoff = cid * NPAD

    def _fire(kk, b):
        pltpu.async_copy(y2.at[src_blk.at[pl.ds(kk * KR * 128, KR * 128)]],
                         rows_v.at[b], semg.at[b])

    def _wait_gathers(b):
        pltpu.make_async_copy(y2.at[pl.ds(0, KR * 128)], rows_v.at[b],
                              semg.at[b]).wait()

    def _scale(kk, b):
        for j in range(KR):
            def _grp(g, _, j=j, b=b):
                wv = w_blk[kk * KR + j, pl.ds(g * 16, 16)]
                for jj in range(16):
                    w = wv[jj]
                    e = j * 128 + g * 16 + jj
                    for q in range(F // 16):
                        sl = pl.ds(q * 16, 16)
                        rows_v[b, e, sl] = rows_v[b, e, sl] * w
                return 0
            lax.fori_loop(0, 8, _grp, 0)

    def _fire_scatter(kk, b):
        for j in range(KR):
            pltpu.async_copy(rows_v.at[b].at[pl.ds(j * 128, 128)],
                             acc_sh.at[dst_blk.at[kk * KR + j]],
                             sems.at[b], add=True)
            pltpu.async_copy(ones_v.at[j], degacc_sh.at[dst_blk.at[kk * KR + j]],
                             sems.at[b], add=True)

    def _wait_scatter(b):
        pltpu.make_async_copy(y2.at[pl.ds(0, KR * 128)], rows_v.at[b],
                              sems.at[b]).wait()
        for j in range(KR):
            pltpu.make_async_copy(wr.at[cid, 0], ones_v.at[j],
                                  sems.at[b]).wait()

    for h in range(4):
        hrow0 = irow0 + h * QB
        pltpu.async_copy(srcs.at[cid, pl.ds(hrow0 * 128, QB * 128)], src_blk,
                         semg.at[0])
        pltpu.async_copy(dsts.at[cid, pl.ds(hrow0, QB)], dst_blk,
                         semg.at[1])
        pltpu.async_copy(wr.at[cid, pl.ds(hrow0, QB)], w_blk, semg.at[2])

        if h == 0:
            for i in range(8):
                ones_v[0, pl.ds(i * 16, 16)] = one16
                ones_v[1, pl.ds(i * 16, 16)] = one16
            _ = 0

            def _z_rows(i, _):
                for b in range(NBUF):
                    for q in range(F // 16):
                        rows_v[b, i, pl.ds(q * 16, 16)] = zero16
                return 0
            lax.fori_loop(0, KR * 128, _z_rows, 0)

            def _z_dacc(i, _):
                deg_acc[pl.ds(i * 16, 16)] = zero16
                return 0
            lax.fori_loop(0, RP // 16, _z_dacc, 0)

            pltpu.sync_copy(rows_v.at[0], acc_sh.at[pl.ds(row0, 256)])
            pltpu.sync_copy(rows_v.at[1], acc_sh.at[pl.ds(row0 + 256, 256)])
            pltpu.sync_copy(rows_v.at[2].at[pl.ds(0, 128)],
                            acc_sh.at[pl.ds(row0 + 512, 128)])
            pltpu.sync_copy(deg_acc, degacc_sh.at[pl.ds(row0, RP)])

        pltpu.make_async_copy(srcs.at[cid, pl.ds(0, QB * 128)], src_blk,
                              semg.at[0]).wait()
        pltpu.make_async_copy(dsts.at[cid, pl.ds(0, QB)], dst_blk,
                              semg.at[1]).wait()
        pltpu.make_async_copy(wr.at[cid, pl.ds(0, QB)], w_blk,
                              semg.at[2]).wait()

        def _off(i, _):
            sl = pl.ds(i * 16, 16)
            src_blk[sl] = src_blk[sl] + yoff
            return 0
        lax.fori_loop(0, QB * 8, _off, 0)

        if h == 0:
            plsc.subcore_barrier()

        for b in range(PRE):
            _fire(b, b)

        def _step(k, _):
            for b in range(NBUF):
                kk = k * NBUF + b
                _wait_gathers(b)
                _scale(kk, b)
                _fire_scatter(kk, b)
                kk2 = kk + PRE
                b2 = (b + PRE) % NBUF

                @pl.when(kk2 < NKH)
                def _():
                    @pl.when(kk2 >= NBUF)
                    def _():
                        _wait_scatter(b2)
                    _fire(kk2, b2)
            return 0

        lax.fori_loop(0, NKH // NBUF, _step, 0)
        for b in range(NBUF):
            _wait_scatter(b)

    plsc.subcore_barrier()
    pltpu.sync_copy(acc_sh.at[pl.ds(row0, 256)], rows_v.at[0])
    pltpu.sync_copy(rows_v.at[0], s_out.at[cid, pl.ds(row0, 256)])
    pltpu.sync_copy(acc_sh.at[pl.ds(row0 + 256, 256)], rows_v.at[1])
    pltpu.sync_copy(rows_v.at[1], s_out.at[cid, pl.ds(row0 + 256, 256)])
    pltpu.sync_copy(acc_sh.at[pl.ds(row0 + 512, 128)],
                    rows_v.at[2].at[pl.ds(0, 128)])
    pltpu.sync_copy(rows_v.at[2].at[pl.ds(0, 128)],
                    s_out.at[cid, pl.ds(row0 + 512, 128)])
    pltpu.sync_copy(degacc_sh.at[pl.ds(row0, RP)], deg_acc)
    pltpu.sync_copy(deg_acc, deg_out.at[cid, pl.ds(row0, RP)])


_sc_call = pl.kernel(
    _sc_body,
    out_type=(jax.ShapeDtypeStruct((NC, NPAD, F), jnp.float32),
              jax.ShapeDtypeStruct((NC, NPAD), jnp.float32)),
    mesh=plsc.VectorSubcoreMesh(core_axis_name="c", subcore_axis_name="s"),
    compiler_params=pltpu.CompilerParams(needs_layout_passes=False,
                                         use_tc_tiling_on_sc=False),
    scratch_types=[
        pltpu.VMEM((QB * 128,), jnp.int32),
        pltpu.VMEM((QB, 128), jnp.int32),
        pltpu.VMEM((QB, 128), jnp.float32),
        pltpu.VMEM((NBUF, KR * 128, F), jnp.float32),
        pltpu.VMEM((KR, 128), jnp.float32),
        pltpu.VMEM((RP,), jnp.float32),
        pltpu.VMEM_SHARED((NPAD, F), jnp.float32),
        pltpu.VMEM_SHARED((NPAD,), jnp.float32),
        pltpu.SemaphoreType.DMA((NBUF,)),
        pltpu.SemaphoreType.DMA((NBUF,)),
    ],
)


def _post_body(z_ref, s_ref, deg_ref, w1_ref, b1_ref, bt_ref, br_ref,
               w2_ref, b2_ref, w3_ref, b3_ref, out_ref):
    c = b1_ref[...] + jnp.dot(bt_ref[...] + br_ref[...], w1_ref[...],
                              precision=_PREC) * THIRD
    d0 = lax.broadcast_in_dim(jnp.maximum(deg_ref[0], 1.0), (NPAD, F), (0,))
    d1 = lax.broadcast_in_dim(jnp.maximum(deg_ref[1], 1.0), (NPAD, F), (0,))
    h1 = z_ref[...] + s_ref[0] / d0 + s_ref[1] / d1 + c
    h1 = jnp.maximum(h1, 0.0)
    h2 = jnp.maximum(jnp.dot(h1, w2_ref[...], precision=_PREC) + b2_ref[...],
                     0.0)
    out = jnp.dot(h2, w3_ref[...], precision=_PREC) + b3_ref[...]
    out_ref[...] = out[:N]


_post_call = pl.pallas_call(
    _post_body,
    out_shape=jax.ShapeDtypeStruct((N, 10), jnp.float32),
)


def kernel(x, edge_index_sim_tic, edge_weight_sim_tic,
           edge_index_related_to, edge_weight_related_to,
           W_self_tic, W_neigh_tic, b_tic,
           W_self_rel, W_neigh_rel, b_rel,
           W1, b1, W2, b2, W3, b3):
    z, y2 = _pre_call(x, W_self_tic, W_neigh_tic, W_self_rel,
                      W_neigh_rel, W1)
    eip = jnp.pad(jnp.stack([edge_index_sim_tic, edge_index_related_to]),
                  ((0, 0), (0, 0), (0, EPAD - E)), constant_values=N)
    srcs = eip[:, 0, :]
    dsts = eip[:, 1, :].reshape(NC, EPR, 128)
    wr = jnp.pad(jnp.stack([edge_weight_sim_tic, edge_weight_related_to]),
                 ((0, 0), (0, EPAD - E))).reshape(NC, EPR, 128)
    s, deg = _sc_call(y2, srcs, dsts, wr)
    out = _post_call(z, s, deg,
                     W1, b1.reshape(1, F), b_tic.reshape(1, D),
                     b_rel.reshape(1, D), W2, b2.reshape(1, 32),
                     W3, b3.reshape(1, 10))
    return (out, out)
